# baseline (speedup 1.0000x reference)
import functools

import jax
import jax.numpy as jnp
from jax import lax
from jax.experimental import pallas as pl
from jax.experimental.pallas import tpu as pltpu
from jax.experimental.pallas import tpu_sc as plsc

_T, _D, _E, _DFF, _NG, _TG, _K, _NS = 2048, 1024, 8, 512, 4, 2, 2, 2
_RSF = 2.5
_NEG = float(jnp.finfo(jnp.float32).min)
_B = 128
_NTM = (_K * _T) // _B + _E
_PPAD = _NTM * _B
_P = _K * _T


def _routing_dispatch_body(x_ref, wg_ref, eb_ref, wsgu_ref, wsd_ref,
                           pos_ref, wexp_ref, eot_ref, sh_ref):
    ti = pl.program_id(0)
    bt = sh_ref.shape[0]
    xblk = x_ref[pl.ds(ti * bt, bt), :]
    gu_s = jnp.dot(xblk, wsgu_ref[...], preferred_element_type=jnp.float32)
    half = _DFF * _NS
    g_s = gu_s[:, :half]
    u_s = gu_s[:, half:]
    h_s = (g_s * jax.nn.sigmoid(g_s)) * u_s
    sh_ref[...] = jnp.dot(h_s, wsd_ref[...], preferred_element_type=jnp.float32)

    @pl.when(ti == 0)
    def _routing():
        _routing_math(x_ref, wg_ref, eb_ref, pos_ref, wexp_ref, eot_ref)


def _routing_math(x_ref, wg_ref, eb_ref, pos_ref, wexp_ref, eot_ref):
    x = x_ref[...]
    logits = jnp.dot(x, wg_ref[...], preferred_element_type=jnp.float32)
    scores = jax.nn.sigmoid(logits)
    sc = scores + eb_ref[...]
    t = x.shape[0]
    ge = lax.broadcasted_iota(jnp.int32, (_E, _NG), 0)
    gg = lax.broadcasted_iota(jnp.int32, (_E, _NG), 1)
    gmat = (ge // (_E // _NG) == gg).astype(jnp.float32)
    sc_h = sc.astype(jnp.bfloat16).astype(jnp.float32)
    sc_m = (sc - sc_h).astype(jnp.bfloat16).astype(jnp.float32)
    sc_l = sc - sc_h - sc_m
    gs = (jnp.dot(sc_h, gmat, preferred_element_type=jnp.float32)
          + jnp.dot(sc_m, gmat, preferred_element_type=jnp.float32)
          + jnp.dot(sc_l, gmat, preferred_element_type=jnp.float32))
    ii4 = lax.broadcasted_iota(jnp.int32, (t, _NG), 1)
    m1 = jnp.max(gs, axis=1, keepdims=True)
    im1 = jnp.min(jnp.where(gs == m1, ii4, _NG), axis=1, keepdims=True)
    gs2 = jnp.where(ii4 == im1, _NEG, gs)
    m2 = jnp.max(gs2, axis=1, keepdims=True)
    im2 = jnp.min(jnp.where(gs2 == m2, ii4, _NG), axis=1, keepdims=True)
    eg = lax.broadcasted_iota(jnp.int32, (t, _E), 1) // (_E // _NG)
    emask = (eg == im1) | (eg == im2)
    masked = jnp.where(emask, sc, _NEG)
    ii8 = lax.broadcasted_iota(jnp.int32, (t, _E), 1)
    mm1 = jnp.max(masked, axis=1, keepdims=True)
    ie1 = jnp.min(jnp.where(masked == mm1, ii8, _E), axis=1, keepdims=True)
    masked2 = jnp.where(ii8 == ie1, _NEG, masked)
    mm2 = jnp.max(masked2, axis=1, keepdims=True)
    ie2 = jnp.min(jnp.where(masked2 == mm2, ii8, _E), axis=1, keepdims=True)
    w1 = jnp.sum(jnp.where(ii8 == ie1, scores, 0.0), axis=1, keepdims=True)
    w2 = jnp.sum(jnp.where(ii8 == ie2, scores, 0.0), axis=1, keepdims=True)
    den = w1 + w2 + 1e-20
    w1n = w1 / den * _RSF
    w2n = w2 / den * _RSF
    wexp_ref[...] = jnp.concatenate(
        [jnp.broadcast_to(w1n, (t, 16)), jnp.broadcast_to(w2n, (t, 16))], axis=0)

    oh1 = (ii8 == ie1).astype(jnp.float32)
    oh2 = (ii8 == ie2).astype(jnp.float32)
    ohp = jnp.concatenate([oh1, oh2], axis=0)
    blk = 1024
    tril = (lax.broadcasted_iota(jnp.int32, (blk, blk), 0)
            > lax.broadcasted_iota(jnp.int32, (blk, blk), 1)).astype(jnp.float32)
    run = jnp.zeros((1, _E), jnp.float32)
    ranks = []
    for b in range(_P // blk):
        obk = ohp[b * blk:(b + 1) * blk]
        r = jnp.dot(tril, obk, preferred_element_type=jnp.float32) + run
        run = run + jnp.sum(obk, axis=0, keepdims=True)
        ranks.append(jnp.sum(r * obk, axis=1, keepdims=True))
    rank = jnp.concatenate(ranks, axis=0)
    counts = run
    tiles_e = jnp.ceil(counts / _B)
    u8 = (lax.broadcasted_iota(jnp.int32, (_E, _E), 0)
          < lax.broadcasted_iota(jnp.int32, (_E, _E), 1)).astype(jnp.float32)
    tile_start = jnp.dot(tiles_e, u8, preferred_element_type=jnp.float32)
    pad_off = tile_start * _B
    padd = jnp.sum(ohp * pad_off, axis=1, keepdims=True)
    pos_ref[...] = (rank + padd).astype(jnp.int32)
    it = lax.broadcasted_iota(jnp.int32, (_NTM, _E), 0).astype(jnp.float32)
    ts_b = jnp.broadcast_to(tile_start, (_NTM, _E))
    eot_ref[...] = (jnp.sum((it >= ts_b).astype(jnp.float32), axis=1,
                            keepdims=True) - 1.0).astype(jnp.int32)


def _ffn_body(eot_ref, xs_ref, wgu_ref, wd_ref, ys_ref):
    del eot_ref
    gu = jnp.dot(xs_ref[...], wgu_ref[0], preferred_element_type=jnp.float32)
    g = gu[:, :_DFF]
    u = gu[:, _DFF:]
    h = (g * jax.nn.sigmoid(g)) * u
    ys_ref[...] = jnp.dot(h, wd_ref[0], preferred_element_type=jnp.float32)


_SC_MESH = dict(core_axis_name="c", subcore_axis_name="s")
_NW = 32
_PPW = _P // _NW
_CH = 16


_GCH = 32
_GNC = _PPW // _GCH


def _sc_gather_call(x, pos2d):
    @functools.partial(
        pl.kernel,
        out_type=jax.ShapeDtypeStruct((_PPAD, _D), jnp.float32),
        mesh=plsc.VectorSubcoreMesh(**_SC_MESH),
        scratch_types=[
            pltpu.VMEM((_GNC, 1, _GCH), jnp.int32),
            pltpu.VMEM((3, _GCH, _D), jnp.float32),
            pltpu.SemaphoreType.DMA,
            pltpu.SemaphoreType.DMA,
            pltpu.SemaphoreType.DMA,
            pltpu.SemaphoreType.DMA,
            pltpu.SemaphoreType.DMA,
            pltpu.SemaphoreType.DMA,
        ],
    )
    def k(x_hbm, pos_hbm, xs_hbm, pos_v, rows_v, g0, g1, g2, s0, s1, s2):
        wid = lax.axis_index("s") * 2 + lax.axis_index("c")
        base = wid * _PPW
        gsem = (g0, g1, g2)
        ssem = (s0, s1, s2)
        pltpu.sync_copy(pos_hbm.at[pl.ds(wid * _GNC, _GNC)], pos_v)

        def gissue(c, b):
            t0 = pl.multiple_of((base + c * _GCH) & (_T - 1), _GCH)
            return pltpu.async_copy(x_hbm.at[pl.ds(t0, _GCH)],
                                    rows_v.at[b], gsem[b])

        gd = [gissue(0, 0), gissue(1, 1), gissue(2, 2)]
        sd = [None] * _GNC
        for c in range(_GNC):
            b = c % 3
            gd[c].wait()
            sd[c] = pltpu.async_copy(
                rows_v.at[b], xs_hbm.at[pos_v.at[c, 0]], ssem[b])
            if c + 3 < _GNC:
                sd[c].wait()
                gd.append(gissue(c + 3, b))
        for c in range(max(_GNC - 3, 0), _GNC):
            sd[c].wait()

    return k(x, pos2d)


_TPW = _T // _NW
_TCH = 16


_CNC = _TPW // _TCH


def _sc_combine_call(ys, pos0_2d, pos1_2d, wexp, shared):
    @functools.partial(
        pl.kernel,
        out_type=jax.ShapeDtypeStruct((_T, _D), jnp.float32),
        mesh=plsc.VectorSubcoreMesh(**_SC_MESH),
        scratch_types=[
            pltpu.VMEM((_CNC, 1, _TCH), jnp.int32),
            pltpu.VMEM((_CNC, 1, _TCH), jnp.int32),
            pltpu.VMEM((_TPW, 16), jnp.float32),
            pltpu.VMEM((_TPW, 16), jnp.float32),
            pltpu.VMEM((2, _TCH, _D), jnp.float32),
            pltpu.VMEM((2, _TCH, _D), jnp.float32),
            pltpu.VMEM((_TCH, _D), jnp.float32),
            pltpu.VMEM((_TCH, _D), jnp.float32),
            pltpu.SemaphoreType.DMA,
            pltpu.SemaphoreType.DMA,
            pltpu.SemaphoreType.DMA,
            pltpu.SemaphoreType.DMA,
        ],
    )
    def k(ys_hbm, pos0_hbm, pos1_hbm, wexp_hbm, sh_hbm, out_hbm,
          pos0_v, pos1_v, w0_v, w1_v, rows0_v, rows1_v, sh_v, out_v,
          ga0, ga1, gb0, gb1):
        wid = lax.axis_index("s") * 2 + lax.axis_index("c")
        base = wid * _TPW
        gsa = (ga0, ga1)
        gsb = (gb0, gb1)
        pltpu.sync_copy(pos0_hbm.at[pl.ds(wid * _CNC, _CNC)], pos0_v)
        pltpu.sync_copy(pos1_hbm.at[pl.ds(wid * _CNC, _CNC)], pos1_v)
        pltpu.sync_copy(wexp_hbm.at[pl.ds(base, _TPW)], w0_v)
        pltpu.sync_copy(wexp_hbm.at[pl.ds(_T + base, _TPW)], w1_v)

        def gissue(c, b):
            return (pltpu.async_copy(ys_hbm.at[pos0_v.at[c, 0]],
                                     rows0_v.at[b], gsa[b]),
                    pltpu.async_copy(ys_hbm.at[pos1_v.at[c, 0]],
                                     rows1_v.at[b], gsb[b]))

        gd = [gissue(0, 0)]
        for c in range(_CNC):
            b = c & 1
            if c + 1 < _CNC:
                gd.append(gissue(c + 1, 1 - b))
            da, db = gd[c]
            da.wait()
            db.wait()
            t0 = base + c * _TCH
            pltpu.sync_copy(sh_hbm.at[pl.ds(t0, _TCH)], sh_v)
            for j in range(_TCH):
                w0j = w0_v[c * _TCH + j]
                w1j = w1_v[c * _TCH + j]

                def body(v, carry, j=j, w0j=w0j, w1j=w1j, b=b):
                    sl = pl.ds(v * 16, 16)
                    out_v[j, sl] = (w0j * rows0_v[b, j, sl]
                                    + w1j * rows1_v[b, j, sl] + sh_v[j, sl])
                    return carry

                lax.fori_loop(0, _D // 16, body, 0, unroll=8)
            pltpu.sync_copy(out_v, out_hbm.at[pl.ds(t0, _TCH)])

    return k(ys, pos0_2d, pos1_2d, wexp, shared)


def kernel(hidden_states, W_gate, e_bias, W_gate_up, W_down, Ws_gate_up, Ws_down):
    x = hidden_states
    eb = e_bias.reshape(1, _E)

    bt2 = 256
    pos2, wexp, eot, shared = pl.pallas_call(
        _routing_dispatch_body,
        grid=(_T // bt2,),
        in_specs=[
            pl.BlockSpec((_T, _D), lambda t: (0, 0)),
            pl.BlockSpec((_D, _E), lambda t: (0, 0)),
            pl.BlockSpec((1, _E), lambda t: (0, 0)),
            pl.BlockSpec((_D, 2 * _DFF * _NS), lambda t: (0, 0)),
            pl.BlockSpec((_DFF * _NS, _D), lambda t: (0, 0)),
        ],
        out_specs=[
            pl.BlockSpec((_P, 1), lambda t: (0, 0)),
            pl.BlockSpec((_P, 16), lambda t: (0, 0)),
            pl.BlockSpec((_NTM, 1), lambda t: (0, 0)),
            pl.BlockSpec((bt2, _D), lambda t: (t, 0)),
        ],
        out_shape=(
            jax.ShapeDtypeStruct((_P, 1), jnp.int32),
            jax.ShapeDtypeStruct((_P, 16), jnp.float32),
            jax.ShapeDtypeStruct((_NTM, 1), jnp.int32),
            jax.ShapeDtypeStruct((_T, _D), jnp.float32),
        ),
    )(x, W_gate, eb, Ws_gate_up, Ws_down)
    pos = pos2.reshape(_P)
    eot1 = eot.reshape(_NTM)

    xs = _sc_gather_call(x, pos.reshape(_NW * _GNC, 1, _GCH))

    ys = pl.pallas_call(
        _ffn_body,
        grid_spec=pltpu.PrefetchScalarGridSpec(
            num_scalar_prefetch=1,
            grid=(_NTM,),
            in_specs=[
                pl.BlockSpec((_B, _D), lambda i, eot: (i, 0)),
                pl.BlockSpec((1, _D, 2 * _DFF), lambda i, eot: (eot[i], 0, 0)),
                pl.BlockSpec((1, _DFF, _D), lambda i, eot: (eot[i], 0, 0)),
            ],
            out_specs=pl.BlockSpec((_B, _D), lambda i, eot: (i, 0)),
        ),
        out_shape=jax.ShapeDtypeStruct((_PPAD, _D), jnp.float32),
    )(eot1, xs, W_gate_up, W_down)

    return _sc_combine_call(ys, pos[:_T].reshape(_NW * _CNC, 1, _TCH),
                            pos[_T:].reshape(_NW * _CNC, 1, _TCH), wexp, shared)


# speedup vs baseline: 1.0317x; 1.0317x over previous
import functools

import jax
import jax.numpy as jnp
from jax import lax
from jax.experimental import pallas as pl
from jax.experimental.pallas import tpu as pltpu
from jax.experimental.pallas import tpu_sc as plsc

_T, _D, _E, _DFF, _NG, _TG, _K, _NS = 2048, 1024, 8, 512, 4, 2, 2, 2
_RSF = 2.5
_NEG = float(jnp.finfo(jnp.float32).min)
_B = 128
_NTM = (_K * _T) // _B + _E
_PPAD = _NTM * _B
_P = _K * _T


def _shared_body(x_ref, wsgu_ref, wsd_ref, out_ref):
    x = x_ref[...]
    gu = jnp.dot(x, wsgu_ref[...], preferred_element_type=jnp.float32)
    half = _DFF * _NS
    g = gu[:, :half]
    u = gu[:, half:]
    h = (g * jax.nn.sigmoid(g)) * u
    out_ref[...] = jnp.dot(h, wsd_ref[...], preferred_element_type=jnp.float32)


def _routing_dispatch_body(x_ref, wg_ref, eb_ref, pos_ref, wexp_ref, eot_ref):
    x = x_ref[...]
    logits = jnp.dot(x, wg_ref[...], preferred_element_type=jnp.float32)
    scores = jax.nn.sigmoid(logits)
    sc = scores + eb_ref[...]
    t = x.shape[0]
    ge = lax.broadcasted_iota(jnp.int32, (_E, _NG), 0)
    gg = lax.broadcasted_iota(jnp.int32, (_E, _NG), 1)
    gmat = (ge // (_E // _NG) == gg).astype(jnp.float32)
    sc_h = sc.astype(jnp.bfloat16).astype(jnp.float32)
    sc_m = (sc - sc_h).astype(jnp.bfloat16).astype(jnp.float32)
    sc_l = sc - sc_h - sc_m
    gs = (jnp.dot(sc_h, gmat, preferred_element_type=jnp.float32)
          + jnp.dot(sc_m, gmat, preferred_element_type=jnp.float32)
          + jnp.dot(sc_l, gmat, preferred_element_type=jnp.float32))
    ii4 = lax.broadcasted_iota(jnp.int32, (t, _NG), 1)
    m1 = jnp.max(gs, axis=1, keepdims=True)
    im1 = jnp.min(jnp.where(gs == m1, ii4, _NG), axis=1, keepdims=True)
    gs2 = jnp.where(ii4 == im1, _NEG, gs)
    m2 = jnp.max(gs2, axis=1, keepdims=True)
    im2 = jnp.min(jnp.where(gs2 == m2, ii4, _NG), axis=1, keepdims=True)
    eg = lax.broadcasted_iota(jnp.int32, (t, _E), 1) // (_E // _NG)
    emask = (eg == im1) | (eg == im2)
    masked = jnp.where(emask, sc, _NEG)
    ii8 = lax.broadcasted_iota(jnp.int32, (t, _E), 1)
    mm1 = jnp.max(masked, axis=1, keepdims=True)
    ie1 = jnp.min(jnp.where(masked == mm1, ii8, _E), axis=1, keepdims=True)
    masked2 = jnp.where(ii8 == ie1, _NEG, masked)
    mm2 = jnp.max(masked2, axis=1, keepdims=True)
    ie2 = jnp.min(jnp.where(masked2 == mm2, ii8, _E), axis=1, keepdims=True)
    w1 = jnp.sum(jnp.where(ii8 == ie1, scores, 0.0), axis=1, keepdims=True)
    w2 = jnp.sum(jnp.where(ii8 == ie2, scores, 0.0), axis=1, keepdims=True)
    den = w1 + w2 + 1e-20
    w1n = w1 / den * _RSF
    w2n = w2 / den * _RSF
    wexp_ref[...] = jnp.concatenate(
        [jnp.broadcast_to(w1n, (t, 16)), jnp.broadcast_to(w2n, (t, 16))], axis=0)

    oh1 = (ii8 == ie1).astype(jnp.float32)
    oh2 = (ii8 == ie2).astype(jnp.float32)
    ohp = jnp.concatenate([oh1, oh2], axis=0)
    blk = 1024
    tril = (lax.broadcasted_iota(jnp.int32, (blk, blk), 0)
            > lax.broadcasted_iota(jnp.int32, (blk, blk), 1)).astype(jnp.float32)
    run = jnp.zeros((1, _E), jnp.float32)
    ranks = []
    for b in range(_P // blk):
        obk = ohp[b * blk:(b + 1) * blk]
        r = jnp.dot(tril, obk, preferred_element_type=jnp.float32) + run
        run = run + jnp.sum(obk, axis=0, keepdims=True)
        ranks.append(jnp.sum(r * obk, axis=1, keepdims=True))
    rank = jnp.concatenate(ranks, axis=0)
    counts = run
    tiles_e = jnp.ceil(counts / _B)
    u8 = (lax.broadcasted_iota(jnp.int32, (_E, _E), 0)
          < lax.broadcasted_iota(jnp.int32, (_E, _E), 1)).astype(jnp.float32)
    tile_start = jnp.dot(tiles_e, u8, preferred_element_type=jnp.float32)
    pad_off = tile_start * _B
    padd = jnp.sum(ohp * pad_off, axis=1, keepdims=True)
    pos_ref[...] = (rank + padd).astype(jnp.int32)
    it = lax.broadcasted_iota(jnp.int32, (_NTM, _E), 0).astype(jnp.float32)
    ts_b = jnp.broadcast_to(tile_start, (_NTM, _E))
    eot_ref[...] = (jnp.sum((it >= ts_b).astype(jnp.float32), axis=1,
                            keepdims=True) - 1.0).astype(jnp.int32)


def _ffn_body(eot_ref, xs_ref, wgu_ref, wd_ref, ys_ref):
    del eot_ref
    gu = jnp.dot(xs_ref[...], wgu_ref[0], preferred_element_type=jnp.float32)
    g = gu[:, :_DFF]
    u = gu[:, _DFF:]
    h = (g * jax.nn.sigmoid(g)) * u
    ys_ref[...] = jnp.dot(h, wd_ref[0], preferred_element_type=jnp.float32)


_SC_MESH = dict(core_axis_name="c", subcore_axis_name="s")
_NW = 32
_PPW = _P // _NW
_CH = 16


_GCH = 32
_GNC = _PPW // _GCH


def _sc_gather_call(x, pos2d):
    @functools.partial(
        pl.kernel,
        out_type=jax.ShapeDtypeStruct((_PPAD, _D), jnp.float32),
        mesh=plsc.VectorSubcoreMesh(**_SC_MESH),
        scratch_types=[
            pltpu.VMEM((_GNC, 1, _GCH), jnp.int32),
            pltpu.VMEM((3, _GCH, _D), jnp.float32),
            pltpu.SemaphoreType.DMA,
            pltpu.SemaphoreType.DMA,
            pltpu.SemaphoreType.DMA,
            pltpu.SemaphoreType.DMA,
            pltpu.SemaphoreType.DMA,
            pltpu.SemaphoreType.DMA,
        ],
    )
    def k(x_hbm, pos_hbm, xs_hbm, pos_v, rows_v, g0, g1, g2, s0, s1, s2):
        wid = lax.axis_index("s") * 2 + lax.axis_index("c")
        base = wid * _PPW
        gsem = (g0, g1, g2)
        ssem = (s0, s1, s2)
        pltpu.sync_copy(pos_hbm.at[pl.ds(wid * _GNC, _GNC)], pos_v)

        def gissue(c, b):
            t0 = pl.multiple_of((base + c * _GCH) & (_T - 1), _GCH)
            return pltpu.async_copy(x_hbm.at[pl.ds(t0, _GCH)],
                                    rows_v.at[b], gsem[b])

        gd = [gissue(0, 0), gissue(1, 1), gissue(2, 2)]
        sd = [None] * _GNC
        for c in range(_GNC):
            b = c % 3
            gd[c].wait()
            sd[c] = pltpu.async_copy(
                rows_v.at[b], xs_hbm.at[pos_v.at[c, 0]], ssem[b])
            if c + 3 < _GNC:
                sd[c].wait()
                gd.append(gissue(c + 3, b))
        for c in range(max(_GNC - 3, 0), _GNC):
            sd[c].wait()

    return k(x, pos2d)


_TPW = _T // _NW
_TCH = 16


_CNC = _TPW // _TCH


def _sc_combine_call(ys, pos0_2d, pos1_2d, wexp, shared):
    @functools.partial(
        pl.kernel,
        out_type=jax.ShapeDtypeStruct((_T, _D), jnp.float32),
        mesh=plsc.VectorSubcoreMesh(**_SC_MESH),
        scratch_types=[
            pltpu.VMEM((_CNC, 1, _TCH), jnp.int32),
            pltpu.VMEM((_CNC, 1, _TCH), jnp.int32),
            pltpu.VMEM((_TPW, 16), jnp.float32),
            pltpu.VMEM((_TPW, 16), jnp.float32),
            pltpu.VMEM((2, _TCH, _D), jnp.float32),
            pltpu.VMEM((2, _TCH, _D), jnp.float32),
            pltpu.VMEM((_TCH, _D), jnp.float32),
            pltpu.VMEM((_TCH, _D), jnp.float32),
            pltpu.SemaphoreType.DMA,
            pltpu.SemaphoreType.DMA,
            pltpu.SemaphoreType.DMA,
            pltpu.SemaphoreType.DMA,
        ],
    )
    def k(ys_hbm, pos0_hbm, pos1_hbm, wexp_hbm, sh_hbm, out_hbm,
          pos0_v, pos1_v, w0_v, w1_v, rows0_v, rows1_v, sh_v, out_v,
          ga0, ga1, gb0, gb1):
        wid = lax.axis_index("s") * 2 + lax.axis_index("c")
        base = wid * _TPW
        gsa = (ga0, ga1)
        gsb = (gb0, gb1)
        pltpu.sync_copy(pos0_hbm.at[pl.ds(wid * _CNC, _CNC)], pos0_v)
        pltpu.sync_copy(pos1_hbm.at[pl.ds(wid * _CNC, _CNC)], pos1_v)
        pltpu.sync_copy(wexp_hbm.at[pl.ds(base, _TPW)], w0_v)
        pltpu.sync_copy(wexp_hbm.at[pl.ds(_T + base, _TPW)], w1_v)

        def gissue(c, b):
            return (pltpu.async_copy(ys_hbm.at[pos0_v.at[c, 0]],
                                     rows0_v.at[b], gsa[b]),
                    pltpu.async_copy(ys_hbm.at[pos1_v.at[c, 0]],
                                     rows1_v.at[b], gsb[b]))

        gd = [gissue(0, 0)]
        for c in range(_CNC):
            b = c & 1
            if c + 1 < _CNC:
                gd.append(gissue(c + 1, 1 - b))
            da, db = gd[c]
            da.wait()
            db.wait()
            t0 = base + c * _TCH
            pltpu.sync_copy(sh_hbm.at[pl.ds(t0, _TCH)], sh_v)
            for j in range(_TCH):
                w0j = w0_v[c * _TCH + j]
                w1j = w1_v[c * _TCH + j]

                def body(v, carry, j=j, w0j=w0j, w1j=w1j, b=b):
                    sl = pl.ds(v * 16, 16)
                    out_v[j, sl] = (w0j * rows0_v[b, j, sl]
                                    + w1j * rows1_v[b, j, sl] + sh_v[j, sl])
                    return carry

                lax.fori_loop(0, _D // 16, body, 0, unroll=8)
            pltpu.sync_copy(out_v, out_hbm.at[pl.ds(t0, _TCH)])

    return k(ys, pos0_2d, pos1_2d, wexp, shared)


def kernel(hidden_states, W_gate, e_bias, W_gate_up, W_down, Ws_gate_up, Ws_down):
    x = hidden_states
    eb = e_bias.reshape(1, _E)

    pos2, wexp, eot = pl.pallas_call(
        _routing_dispatch_body,
        out_shape=(
            jax.ShapeDtypeStruct((_P, 1), jnp.int32),
            jax.ShapeDtypeStruct((_P, 16), jnp.float32),
            jax.ShapeDtypeStruct((_NTM, 1), jnp.int32),
        ),
    )(x, W_gate, eb)
    pos = pos2.reshape(_P)
    eot1 = eot.reshape(_NTM)

    bt2 = 256
    shared = pl.pallas_call(
        _shared_body,
        grid=(_T // bt2,),
        in_specs=[
            pl.BlockSpec((bt2, _D), lambda t: (t, 0)),
            pl.BlockSpec((_D, 2 * _DFF * _NS), lambda t: (0, 0)),
            pl.BlockSpec((_DFF * _NS, _D), lambda t: (0, 0)),
        ],
        out_specs=pl.BlockSpec((bt2, _D), lambda t: (t, 0)),
        out_shape=jax.ShapeDtypeStruct((_T, _D), jnp.float32),
    )(x, Ws_gate_up, Ws_down)

    xs = _sc_gather_call(x, pos.reshape(_NW * _GNC, 1, _GCH))

    ys = pl.pallas_call(
        _ffn_body,
        grid_spec=pltpu.PrefetchScalarGridSpec(
            num_scalar_prefetch=1,
            grid=(_NTM,),
            in_specs=[
                pl.BlockSpec((_B, _D), lambda i, eot: (i, 0)),
                pl.BlockSpec((1, _D, 2 * _DFF), lambda i, eot: (eot[i], 0, 0)),
                pl.BlockSpec((1, _DFF, _D), lambda i, eot: (eot[i], 0, 0)),
            ],
            out_specs=pl.BlockSpec((_B, _D), lambda i, eot: (i, 0)),
        ),
        out_shape=jax.ShapeDtypeStruct((_PPAD, _D), jnp.float32),
    )(eot1, xs, W_gate_up, W_down)

    return _sc_combine_call(ys, pos[:_T].reshape(_NW * _CNC, 1, _TCH),
                            pos[_T:].reshape(_NW * _CNC, 1, _TCH), wexp, shared)
